# baseline (speedup 1.0000x reference)
import functools

import jax
import jax.numpy as jnp
from jax import lax
from jax.experimental import pallas as pl
from jax.experimental.pallas import tpu as pltpu
from jax.experimental.pallas import tpu_sc as plsc

NE = 8
DM = 1024
DF = 4096
TT = 8192
BASE_CAP = TT // NE
CAPMAX = BASE_CAP + BASE_CAP // 2
NB = NE * CAPMAX
TRASH = NB
NBPAD = NB + 8
CH = 256
NCH = TT // CH
OVW = 512
NEG = -1e30


LCH = 1024
NLCH = TT // LCH


def _transpose_row(col, n):
    rr = lax.broadcasted_iota(jnp.int32, (n, n), 0)
    cc = lax.broadcasted_iota(jnp.int32, (n, n), 1)
    eye = (rr == cc).astype(jnp.float32)
    return lax.dot_general(col, eye, (((0,), (0,)), ((), ())),
                           preferred_element_type=jnp.float32)


def _row_of(col8):
    rr = lax.broadcasted_iota(jnp.int32, (NE, NE), 0)
    cc = lax.broadcasted_iota(jnp.int32, (NE, NE), 1)
    eye = (rr == cc).astype(jnp.float32)
    return jnp.sum(col8 * eye, axis=0, keepdims=True)


def _router_body(x_ref, w_ref, pmax_ref, assign_ref, slot_ref, cnt_ref,
                 lb_ref, sb_ref, ovf_ref, rank_ref):
    x = x_ref[...]
    w = w_ref[...]
    logits = lax.dot_general(w, x, (((1,), (1,)), ((), ())),
                             preferred_element_type=jnp.float32)
    m = jnp.max(logits, axis=0, keepdims=True)
    ex = jnp.exp(logits - m)
    s = jnp.sum(ex, axis=0, keepdims=True)
    probs = ex / s
    pm = jnp.max(probs, axis=0, keepdims=True)
    pmax_ref[...] = pm

    iota8c = lax.broadcasted_iota(jnp.int32, (NE, TT), 0)
    routes = jnp.min(jnp.where(probs == pm, iota8c, NE), axis=0, keepdims=True)
    oh = (routes == iota8c).astype(jnp.float32)

    rI = lax.broadcasted_iota(jnp.int32, (CH, CH), 0)
    cI = lax.broadcasted_iota(jnp.int32, (CH, CH), 1)
    triu = (rI <= cI).astype(jnp.float32)

    def cumsum_rank(oh_arr):
        o = jnp.zeros((NE, 1), jnp.float32)
        for k in range(NCH):
            blk = lax.slice(oh_arr, (0, k * CH), (NE, (k + 1) * CH))
            cs = lax.dot_general(blk, triu, (((1,), (0,)), ((), ())),
                                 preferred_element_type=jnp.float32) + o
            rank_ref[:, k * CH:(k + 1) * CH] = jnp.sum(
                cs * blk, axis=0, keepdims=True)
            o = lax.slice(cs, (0, CH - 1), (NE, CH))
        return o

    cnt0_col = cumsum_rank(oh)
    rank = rank_ref[...] - 1.0

    delta = jnp.clip((cnt0_col - jnp.float32(BASE_CAP)) * jnp.float32(0.2),
                     jnp.float32(0.0), jnp.float32(BASE_CAP * 0.5))
    cap_col = jnp.float32(BASE_CAP) + delta.astype(jnp.int32).astype(jnp.float32)
    cap_tok = jnp.sum(oh * cap_col, axis=0, keepdims=True)
    kept = rank < cap_tok

    cnt_trim_col = jnp.minimum(cnt0_col, cap_col)
    spare_row = _row_of(cap_col - cnt_trim_col)
    cntf_row = _row_of(cnt_trim_col)
    ovf_cnt_col = cnt0_col - cnt_trim_col
    ovf_cnt_row = _row_of(ovf_cnt_col)
    r8 = lax.broadcasted_iota(jnp.int32, (NE, NE), 0)
    c8 = lax.broadcasted_iota(jnp.int32, (NE, NE), 1)
    sl8 = (c8 < r8).astype(jnp.float32)
    ovf_off_col = jnp.sum(ovf_cnt_row * sl8, axis=1, keepdims=True)
    n_ovf = jnp.sum(ovf_cnt_col).astype(jnp.int32)

    p_t = jnp.sum(oh * ovf_off_col, axis=0, keepdims=True) + rank - cap_tok
    p_t = jnp.where(kept, jnp.float32(-1.0), p_t)

    n_tiles = (n_ovf + (OVW - 1)) // OVW
    iota_col = lax.broadcasted_iota(jnp.int32, (OVW, 1), 0).astype(jnp.float32)

    def build_tile(j, carry):
        base = (j * OVW).astype(jnp.float32)
        tgt = base + iota_col
        accs = [jnp.zeros((OVW, 1), jnp.float32) for _ in range(NE)]
        for c in range(NLCH):
            pc = lax.slice(p_t, (0, c * LCH), (1, (c + 1) * LCH))
            hit = (tgt == pc).astype(jnp.float32)
            for e in range(NE):
                pe = lax.slice(probs, (e, c * LCH), (e + 1, (c + 1) * LCH))
                accs[e] = accs[e] + jnp.sum(hit * pe, axis=1, keepdims=True)
        zero8 = jnp.zeros((OVW, NE), jnp.float32)
        res = jnp.concatenate(accs + [zero8], axis=1)
        ovf_ref[pl.ds(j * OVW, OVW), :] = res
        return carry

    lax.fori_loop(0, n_tiles, build_tile, 0)

    iota8r = lax.broadcasted_iota(jnp.int32, (1, NE), 1)
    iota16r = lax.broadcasted_iota(jnp.int32, (1, 16), 1)

    def greedy(i, carry):
        spare, cntf = carry
        row = ovf_ref[pl.ds(i, 1), :]
        prow = lax.slice(row, (0, 0), (1, NE))
        cand = prow / (1.0 + cntf)
        avail = spare > 0.5
        masked = jnp.where(avail, cand, NEG)
        mx = jnp.max(masked)
        bj = jnp.min(jnp.where(masked == mx, iota8r, NE))
        do = jnp.any(avail)
        upd = jnp.where((iota8r == bj) & do, jnp.float32(1.0), jnp.float32(0.0))
        bj_store = jnp.where(do, bj.astype(jnp.float32), jnp.float32(-1.0))
        ovf_ref[pl.ds(i, 1), :] = jnp.where(iota16r == NE, bj_store, row)
        return spare - upd, cntf + upd

    spare_f, cnt_f = lax.fori_loop(0, n_ovf, greedy, (spare_row, cntf_row))
    cnt_ref[...] = cnt_f

    def recon_tile(j, bjg):
        rows = ovf_ref[pl.ds(j * OVW, OVW), :]
        bj_col = jnp.sum(jnp.where(iota16r == NE, rows, 0.0),
                         axis=1, keepdims=True)
        bj_row = _transpose_row(bj_col, OVW)
        base = (j * OVW).astype(jnp.float32)
        tgt = base + iota_col
        pieces = []
        for c in range(NLCH):
            pc = lax.slice(p_t, (0, c * LCH), (1, (c + 1) * LCH))
            hit = (tgt == pc).astype(jnp.float32)
            pieces.append(lax.dot_general(
                bj_row, hit, (((1,), (0,)), ((), ())),
                preferred_element_type=jnp.float32))
        return bjg + jnp.concatenate(pieces, axis=1)

    bjg = lax.fori_loop(0, n_tiles, recon_tile, jnp.zeros((1, TT), jnp.float32))
    assign = jnp.where(kept, routes, bjg.astype(jnp.int32))
    assign_ref[...] = assign

    p_sum_col = jnp.sum(probs, axis=1, keepdims=True)
    eye8 = (r8 == c8).astype(jnp.float32)
    lb = jnp.sum(cnt_f * p_sum_col * eye8) * jnp.float32(0.01 * NE / (TT * float(TT)))
    lb_ref[...] = jnp.full((1, 1), 1.0, jnp.float32) * lb
    g = lax.dot_general(w, w, (((1,), (1,)), ((), ())),
                        preferred_element_type=jnp.float32)
    goff = g * (1.0 - eye8)
    sb_ref[...] = jnp.full((1, 1), 1.0, jnp.float32) * (
        jnp.sum(goff * goff) * jnp.float32(0.001))

    oh2 = (assign == iota8c).astype(jnp.float32)
    cumsum_rank(oh2)
    rank2 = rank_ref[...] - 1.0
    slot = assign * CAPMAX + rank2.astype(jnp.int32)
    slot_ref[...] = jnp.where(assign >= 0, slot, TRASH)


def _router(tokens, w_switch):
    return pl.pallas_call(
        _router_body,
        out_shape=[
            jax.ShapeDtypeStruct((1, TT), jnp.float32),
            jax.ShapeDtypeStruct((1, TT), jnp.int32),
            jax.ShapeDtypeStruct((1, TT), jnp.int32),
            jax.ShapeDtypeStruct((1, NE), jnp.float32),
            jax.ShapeDtypeStruct((1, 1), jnp.float32),
            jax.ShapeDtypeStruct((1, 1), jnp.float32),
        ],
        scratch_shapes=[
            pltpu.VMEM((TT, 16), jnp.float32),
            pltpu.VMEM((1, TT), jnp.float32),
        ],
    )(tokens, w_switch)


SC_NC = 2
SC_NS = 16


def _make_sc_dispatch():
    nw = SC_NC * SC_NS
    per_w = TT // nw
    chunk = 64
    nchunk = per_w // chunk
    mesh = plsc.VectorSubcoreMesh(core_axis_name="c", subcore_axis_name="s",
                                  num_cores=SC_NC, num_subcores=SC_NS)

    @functools.partial(
        pl.kernel, mesh=mesh,
        out_type=jax.ShapeDtypeStruct((NBPAD, DM), jnp.float32),
        scratch_types=[
            pltpu.VMEM((chunk,), jnp.int32),
            pltpu.VMEM((chunk, DM), jnp.float32),
            pltpu.SemaphoreType.DMA,
        ],
    )
    def dispatch(tok_hbm, slot_hbm, out_hbm, idx_v, rows_v, sem):
        wid = lax.axis_index("s") * SC_NC + lax.axis_index("c")
        base = wid * per_w
        for k in range(nchunk):
            b = base + k * chunk
            pltpu.sync_copy(slot_hbm.at[pl.ds(b, chunk)], idx_v)
            pltpu.sync_copy(tok_hbm.at[pl.ds(b, chunk)], rows_v)
            pltpu.async_copy(rows_v, out_hbm.at[idx_v], sem).wait()

    return dispatch


def _make_sc_combine():
    nw = SC_NC * SC_NS
    per_w = TT // nw
    chunk = 64
    nchunk = per_w // chunk
    mesh = plsc.VectorSubcoreMesh(core_axis_name="c", subcore_axis_name="s",
                                  num_cores=SC_NC, num_subcores=SC_NS)

    @functools.partial(
        pl.kernel, mesh=mesh,
        out_type=jax.ShapeDtypeStruct((TT, DM), jnp.float32),
        scratch_types=[
            pltpu.VMEM((chunk,), jnp.int32),
            pltpu.VMEM((chunk, DM), jnp.float32),
            pltpu.SemaphoreType.DMA,
        ],
    )
    def combine(ybuck_hbm, slot_hbm, out_hbm, idx_v, rows_v, sem):
        wid = lax.axis_index("s") * SC_NC + lax.axis_index("c")
        base = wid * per_w
        for k in range(nchunk):
            b = base + k * chunk
            pltpu.sync_copy(slot_hbm.at[pl.ds(b, chunk)], idx_v)
            pltpu.async_copy(ybuck_hbm.at[idx_v], rows_v, sem).wait()
            pltpu.sync_copy(rows_v, out_hbm.at[pl.ds(b, chunk)])

    return combine


FFT = 512
NFF = DF // FFT


def _ffn_body(xb_ref, w1_ref, b1_ref, w2_ref, b2_ref, out_ref):
    f = pl.program_id(1)
    xb = xb_ref[...].astype(jnp.bfloat16)
    h = lax.dot_general(xb, w1_ref[0], (((1,), (1,)), ((), ())),
                        preferred_element_type=jnp.float32)
    h = jnp.maximum(h + b1_ref[0], 0.0).astype(jnp.bfloat16)
    y = lax.dot_general(h, w2_ref[0], (((1,), (1,)), ((), ())),
                        preferred_element_type=jnp.float32)

    @pl.when(f == 0)
    def _():
        out_ref[...] = y + b2_ref[0]

    @pl.when(f != 0)
    def _():
        out_ref[...] = out_ref[...] + y


def _ffn(buckets, w1, b1, w2, b2):
    return pl.pallas_call(
        _ffn_body,
        grid=(NE, NFF),
        in_specs=[
            pl.BlockSpec((CAPMAX, DM), lambda e, f: (e, 0)),
            pl.BlockSpec((1, FFT, DM), lambda e, f: (e, f, 0)),
            pl.BlockSpec((1, 1, FFT), lambda e, f: (e, 0, f)),
            pl.BlockSpec((1, DM, FFT), lambda e, f: (e, 0, f)),
            pl.BlockSpec((1, 1, DM), lambda e, f: (e, 0, 0)),
        ],
        out_specs=pl.BlockSpec((CAPMAX, DM), lambda e, f: (e, 0)),
        out_shape=jax.ShapeDtypeStruct((NBPAD, DM), jnp.float32),
    )(buckets, w1.astype(jnp.bfloat16), b1.reshape(NE, 1, DF),
      w2.astype(jnp.bfloat16), b2.reshape(NE, 1, DM))


TOKT = 512


def _combine_body(x_ref, g_ref, a_ref, p_ref, o_ref):
    drop = a_ref[...] < 0
    o_ref[...] = jnp.where(drop, x_ref[...], g_ref[...]) * p_ref[...]


def _final_combine(tokens, gathered, assign, pmax):
    return pl.pallas_call(
        _combine_body,
        grid=(TT // TOKT,),
        in_specs=[
            pl.BlockSpec((TOKT, DM), lambda i: (i, 0)),
            pl.BlockSpec((TOKT, DM), lambda i: (i, 0)),
            pl.BlockSpec((TOKT, 1), lambda i: (i, 0)),
            pl.BlockSpec((TOKT, 1), lambda i: (i, 0)),
        ],
        out_specs=pl.BlockSpec((TOKT, DM), lambda i: (i, 0)),
        out_shape=jax.ShapeDtypeStruct((TT, DM), jnp.float32),
    )(tokens, gathered, assign, pmax)


_sc_dispatch = _make_sc_dispatch()
_sc_combine = _make_sc_combine()


def kernel(x, W_switch, W1, b1, W2, b2):
    S, B, D = x.shape
    tokens = x.reshape(TT, DM)
    pmax, assign, slot, _cnt, lb, sb = _router(tokens, W_switch)
    slot1d = slot.reshape(TT)
    buckets = _sc_dispatch(tokens, slot1d)
    ybuck = _ffn(buckets, W1, b1, W2, b2)
    gathered = _sc_combine(ybuck, slot1d)
    out = _final_combine(tokens, gathered, assign.reshape(TT, 1), pmax.reshape(TT, 1))
    return out.reshape(S, B, D), lb.reshape(()), sb.reshape(())


# speedup vs baseline: 1.2167x; 1.2167x over previous
import functools

import jax
import jax.numpy as jnp
from jax import lax
from jax.experimental import pallas as pl
from jax.experimental.pallas import tpu as pltpu
from jax.experimental.pallas import tpu_sc as plsc

NE = 8
DM = 1024
DF = 4096
TT = 8192
BASE_CAP = TT // NE
CAPMAX = BASE_CAP + BASE_CAP // 2
NB = NE * CAPMAX
TRASH = NB
NBPAD = NB + 8
CH = 256
NCH = TT // CH
OVW = 512
NEG = -1e30


LCH = 1024
NLCH = TT // LCH


def _transpose_row(col, n):
    rr = lax.broadcasted_iota(jnp.int32, (n, n), 0)
    cc = lax.broadcasted_iota(jnp.int32, (n, n), 1)
    eye = (rr == cc).astype(jnp.float32)
    return lax.dot_general(col, eye, (((0,), (0,)), ((), ())),
                           preferred_element_type=jnp.float32)


def _row_of(col8):
    rr = lax.broadcasted_iota(jnp.int32, (NE, NE), 0)
    cc = lax.broadcasted_iota(jnp.int32, (NE, NE), 1)
    eye = (rr == cc).astype(jnp.float32)
    return jnp.sum(col8 * eye, axis=0, keepdims=True)


def _router_body(x_ref, w_ref, pmax_ref, assign_ref, slot_ref, cnt_ref,
                 lb_ref, sb_ref, ovf_ref, rank_ref):
    x = x_ref[...]
    w = w_ref[...]
    logits = lax.dot_general(w, x, (((1,), (1,)), ((), ())),
                             preferred_element_type=jnp.float32)
    m = jnp.max(logits, axis=0, keepdims=True)
    ex = jnp.exp(logits - m)
    s = jnp.sum(ex, axis=0, keepdims=True)
    probs = ex / s
    pm = jnp.max(probs, axis=0, keepdims=True)
    pmax_ref[...] = pm

    iota8c = lax.broadcasted_iota(jnp.int32, (NE, TT), 0)
    routes = jnp.min(jnp.where(probs == pm, iota8c, NE), axis=0, keepdims=True)
    oh = (routes == iota8c).astype(jnp.float32)

    rI = lax.broadcasted_iota(jnp.int32, (CH, CH), 0)
    cI = lax.broadcasted_iota(jnp.int32, (CH, CH), 1)
    triu = (rI <= cI).astype(jnp.float32)

    def cumsum_rank(oh_arr):
        o = jnp.zeros((NE, 1), jnp.float32)
        for k in range(NCH):
            blk = lax.slice(oh_arr, (0, k * CH), (NE, (k + 1) * CH))
            cs = lax.dot_general(blk, triu, (((1,), (0,)), ((), ())),
                                 preferred_element_type=jnp.float32) + o
            rank_ref[:, k * CH:(k + 1) * CH] = jnp.sum(
                cs * blk, axis=0, keepdims=True)
            o = lax.slice(cs, (0, CH - 1), (NE, CH))
        return o

    cnt0_col = cumsum_rank(oh)
    rank = rank_ref[...] - 1.0

    delta = jnp.clip((cnt0_col - jnp.float32(BASE_CAP)) * jnp.float32(0.2),
                     jnp.float32(0.0), jnp.float32(BASE_CAP * 0.5))
    cap_col = jnp.float32(BASE_CAP) + delta.astype(jnp.int32).astype(jnp.float32)
    cap_tok = jnp.sum(oh * cap_col, axis=0, keepdims=True)
    kept = rank < cap_tok

    cnt_trim_col = jnp.minimum(cnt0_col, cap_col)
    spare_row = _row_of(cap_col - cnt_trim_col)
    cntf_row = _row_of(cnt_trim_col)
    ovf_cnt_col = cnt0_col - cnt_trim_col
    ovf_cnt_row = _row_of(ovf_cnt_col)
    r8 = lax.broadcasted_iota(jnp.int32, (NE, NE), 0)
    c8 = lax.broadcasted_iota(jnp.int32, (NE, NE), 1)
    sl8 = (c8 < r8).astype(jnp.float32)
    ovf_off_col = jnp.sum(ovf_cnt_row * sl8, axis=1, keepdims=True)
    n_ovf = jnp.sum(ovf_cnt_col).astype(jnp.int32)

    p_t = jnp.sum(oh * ovf_off_col, axis=0, keepdims=True) + rank - cap_tok
    p_t = jnp.where(kept, jnp.float32(-1.0), p_t)

    n_tiles = (n_ovf + (OVW - 1)) // OVW
    iota_col = lax.broadcasted_iota(jnp.int32, (OVW, 1), 0).astype(jnp.float32)

    def build_tile(j, carry):
        base = (j * OVW).astype(jnp.float32)
        tgt = base + iota_col
        accs = [jnp.zeros((OVW, 1), jnp.float32) for _ in range(NE)]
        for c in range(NLCH):
            pc = lax.slice(p_t, (0, c * LCH), (1, (c + 1) * LCH))
            hit = (tgt == pc).astype(jnp.float32)
            for e in range(NE):
                pe = lax.slice(probs, (e, c * LCH), (e + 1, (c + 1) * LCH))
                accs[e] = accs[e] + jnp.sum(hit * pe, axis=1, keepdims=True)
        zero8 = jnp.zeros((OVW, NE), jnp.float32)
        res = jnp.concatenate(accs + [zero8], axis=1)
        ovf_ref[pl.ds(j * OVW, OVW), :] = res
        return carry

    lax.fori_loop(0, n_tiles, build_tile, 0)

    iota8r = lax.broadcasted_iota(jnp.int32, (1, NE), 1)
    iota16r = lax.broadcasted_iota(jnp.int32, (1, 16), 1)

    def greedy(i, carry):
        spare, cntf = carry
        row = ovf_ref[pl.ds(i, 1), :]
        prow = lax.slice(row, (0, 0), (1, NE))
        cand = prow / (1.0 + cntf)
        avail = spare > 0.5
        masked = jnp.where(avail, cand, NEG)
        mx = jnp.max(masked)
        bj = jnp.min(jnp.where(masked == mx, iota8r, NE))
        do = jnp.any(avail)
        upd = jnp.where((iota8r == bj) & do, jnp.float32(1.0), jnp.float32(0.0))
        bj_store = jnp.where(do, bj.astype(jnp.float32), jnp.float32(-1.0))
        ovf_ref[pl.ds(i, 1), :] = jnp.where(iota16r == NE, bj_store, row)
        return spare - upd, cntf + upd

    spare_f, cnt_f = lax.fori_loop(0, n_ovf, greedy, (spare_row, cntf_row))
    cnt_ref[...] = cnt_f

    def recon_tile(j, bjg):
        rows = ovf_ref[pl.ds(j * OVW, OVW), :]
        bj_col = jnp.sum(jnp.where(iota16r == NE, rows, 0.0),
                         axis=1, keepdims=True)
        bj_row = _transpose_row(bj_col, OVW)
        base = (j * OVW).astype(jnp.float32)
        tgt = base + iota_col
        pieces = []
        for c in range(NLCH):
            pc = lax.slice(p_t, (0, c * LCH), (1, (c + 1) * LCH))
            hit = (tgt == pc).astype(jnp.float32)
            pieces.append(lax.dot_general(
                bj_row, hit, (((1,), (0,)), ((), ())),
                preferred_element_type=jnp.float32))
        return bjg + jnp.concatenate(pieces, axis=1)

    bjg = lax.fori_loop(0, n_tiles, recon_tile, jnp.zeros((1, TT), jnp.float32))
    assign = jnp.where(kept, routes, bjg.astype(jnp.int32))
    assign_ref[...] = assign

    p_sum_col = jnp.sum(probs, axis=1, keepdims=True)
    eye8 = (r8 == c8).astype(jnp.float32)
    lb = jnp.sum(cnt_f * p_sum_col * eye8) * jnp.float32(0.01 * NE / (TT * float(TT)))
    lb_ref[...] = jnp.full((1, 1), 1.0, jnp.float32) * lb
    g = lax.dot_general(w, w, (((1,), (1,)), ((), ())),
                        preferred_element_type=jnp.float32)
    goff = g * (1.0 - eye8)
    sb_ref[...] = jnp.full((1, 1), 1.0, jnp.float32) * (
        jnp.sum(goff * goff) * jnp.float32(0.001))

    oh2 = (assign == iota8c).astype(jnp.float32)
    cumsum_rank(oh2)
    rank2 = rank_ref[...] - 1.0
    slot = assign * CAPMAX + rank2.astype(jnp.int32)
    slot_ref[...] = jnp.where(assign >= 0, slot, TRASH)


def _router(tokens, w_switch):
    return pl.pallas_call(
        _router_body,
        out_shape=[
            jax.ShapeDtypeStruct((1, TT), jnp.float32),
            jax.ShapeDtypeStruct((1, TT), jnp.int32),
            jax.ShapeDtypeStruct((1, TT), jnp.int32),
            jax.ShapeDtypeStruct((1, NE), jnp.float32),
            jax.ShapeDtypeStruct((1, 1), jnp.float32),
            jax.ShapeDtypeStruct((1, 1), jnp.float32),
        ],
        scratch_shapes=[
            pltpu.VMEM((TT, 16), jnp.float32),
            pltpu.VMEM((1, TT), jnp.float32),
        ],
    )(tokens, w_switch)


SC_NC = 2
SC_NS = 16


def _make_sc_dispatch():
    nw = SC_NC * SC_NS
    per_w = TT // nw
    chunk = 64
    nchunk = per_w // chunk
    mesh = plsc.VectorSubcoreMesh(core_axis_name="c", subcore_axis_name="s",
                                  num_cores=SC_NC, num_subcores=SC_NS)

    @functools.partial(
        pl.kernel, mesh=mesh,
        out_type=jax.ShapeDtypeStruct((NBPAD, DM), jnp.float32),
        scratch_types=[
            pltpu.VMEM((chunk,), jnp.int32),
            pltpu.VMEM((chunk, DM), jnp.float32),
            pltpu.SemaphoreType.DMA,
        ],
    )
    def dispatch(tok_hbm, slot_hbm, out_hbm, idx_v, rows_v, sem):
        wid = lax.axis_index("s") * SC_NC + lax.axis_index("c")
        base = wid * per_w
        for k in range(nchunk):
            b = base + k * chunk
            pltpu.sync_copy(slot_hbm.at[pl.ds(b, chunk)], idx_v)
            pltpu.sync_copy(tok_hbm.at[pl.ds(b, chunk)], rows_v)
            pltpu.async_copy(rows_v, out_hbm.at[idx_v], sem).wait()

    return dispatch


def _make_sc_combine():
    nw = SC_NC * SC_NS
    per_w = TT // nw
    chunk = 64
    nchunk = per_w // chunk
    mesh = plsc.VectorSubcoreMesh(core_axis_name="c", subcore_axis_name="s",
                                  num_cores=SC_NC, num_subcores=SC_NS)

    @functools.partial(
        pl.kernel, mesh=mesh,
        out_type=jax.ShapeDtypeStruct((TT, DM), jnp.float32),
        scratch_types=[
            pltpu.VMEM((chunk,), jnp.int32),
            pltpu.VMEM((chunk, DM), jnp.float32),
            pltpu.SemaphoreType.DMA,
        ],
    )
    def combine(ybuck_hbm, slot_hbm, out_hbm, idx_v, rows_v, sem):
        wid = lax.axis_index("s") * SC_NC + lax.axis_index("c")
        base = wid * per_w
        for k in range(nchunk):
            b = base + k * chunk
            pltpu.sync_copy(slot_hbm.at[pl.ds(b, chunk)], idx_v)
            pltpu.async_copy(ybuck_hbm.at[idx_v], rows_v, sem).wait()
            pltpu.sync_copy(rows_v, out_hbm.at[pl.ds(b, chunk)])

    return combine


FFT = 512
NFF = DF // FFT


def _ffn_body(xb_ref, w1_ref, b1_ref, w2_ref, b2_ref, out_ref):
    f = pl.program_id(1)
    xb = xb_ref[...].astype(jnp.bfloat16)
    h = lax.dot_general(xb, w1_ref[0].astype(jnp.bfloat16),
                        (((1,), (1,)), ((), ())),
                        preferred_element_type=jnp.float32)
    h = jnp.maximum(h + b1_ref[0], 0.0).astype(jnp.bfloat16)
    y = lax.dot_general(h, w2_ref[0].astype(jnp.bfloat16),
                        (((1,), (1,)), ((), ())),
                        preferred_element_type=jnp.float32)

    @pl.when(f == 0)
    def _():
        out_ref[...] = y + b2_ref[0]

    @pl.when(f != 0)
    def _():
        out_ref[...] = out_ref[...] + y


def _ffn(buckets, w1, b1, w2, b2):
    return pl.pallas_call(
        _ffn_body,
        grid=(NE, NFF),
        in_specs=[
            pl.BlockSpec((CAPMAX, DM), lambda e, f: (e, 0)),
            pl.BlockSpec((1, FFT, DM), lambda e, f: (e, f, 0)),
            pl.BlockSpec((1, 1, FFT), lambda e, f: (e, 0, f)),
            pl.BlockSpec((1, DM, FFT), lambda e, f: (e, 0, f)),
            pl.BlockSpec((1, 1, DM), lambda e, f: (e, 0, 0)),
        ],
        out_specs=pl.BlockSpec((CAPMAX, DM), lambda e, f: (e, 0)),
        out_shape=jax.ShapeDtypeStruct((NBPAD, DM), jnp.float32),
    )(buckets, w1, b1.reshape(NE, 1, DF), w2, b2.reshape(NE, 1, DM))


TOKT = 512


def _combine_body(x_ref, g_ref, a_ref, p_ref, o_ref):
    drop = a_ref[...] < 0
    o_ref[...] = jnp.where(drop, x_ref[...], g_ref[...]) * p_ref[...]


def _final_combine(tokens, gathered, assign, pmax):
    return pl.pallas_call(
        _combine_body,
        grid=(TT // TOKT,),
        in_specs=[
            pl.BlockSpec((TOKT, DM), lambda i: (i, 0)),
            pl.BlockSpec((TOKT, DM), lambda i: (i, 0)),
            pl.BlockSpec((TOKT, 1), lambda i: (i, 0)),
            pl.BlockSpec((TOKT, 1), lambda i: (i, 0)),
        ],
        out_specs=pl.BlockSpec((TOKT, DM), lambda i: (i, 0)),
        out_shape=jax.ShapeDtypeStruct((TT, DM), jnp.float32),
    )(tokens, gathered, assign, pmax)


_sc_dispatch = _make_sc_dispatch()
_sc_combine = _make_sc_combine()


def kernel(x, W_switch, W1, b1, W2, b2):
    S, B, D = x.shape
    tokens = x.reshape(TT, DM)
    pmax, assign, slot, _cnt, lb, sb = _router(tokens, W_switch)
    slot1d = slot.reshape(TT)
    buckets = _sc_dispatch(tokens, slot1d)
    ybuck = _ffn(buckets, W1, b1, W2, b2)
    gathered = _sc_combine(ybuck, slot1d)
    out = _final_combine(tokens, gathered, assign.reshape(TT, 1), pmax.reshape(TT, 1))
    return out.reshape(S, B, D), lb.reshape(()), sb.reshape(())
